# baseline (speedup 1.0000x reference)
import functools

import jax
import jax.numpy as jnp
from jax import lax
from jax.experimental import pallas as pl
from jax.experimental.pallas import tpu as pltpu
from jax.experimental.pallas import tpu_sc as plsc

N = 10000
NPAD = 10240
NC = 2
NS = 16
BN = 512


def _dense1_body(x_ref, w_ref, al_ref, ar_ref, z_ref, ss_ref, sd_ref):
    z = jax.lax.dot_general(x_ref[...], w_ref[...], (((1,), (1,)), ((), ())),
                            preferred_element_type=jnp.float32)
    z_ref[...] = jnp.stack([z[:, h * 64:(h + 1) * 64] for h in range(4)],
                           axis=0)
    zh = z.reshape(z.shape[0], 4, 64)
    ss_ref[...] = jnp.sum(zh * al_ref[...][None], axis=2)
    sd_ref[...] = jnp.sum(zh * ar_ref[...][None], axis=2)


def _dense1(xp, w1f, a1l, a1r):
    return pl.pallas_call(
        _dense1_body,
        grid=(NPAD // BN,),
        in_specs=[
            pl.BlockSpec((BN, 128), lambda i: (i, 0)),
            pl.BlockSpec((256, 128), lambda i: (0, 0)),
            pl.BlockSpec((4, 64), lambda i: (0, 0)),
            pl.BlockSpec((4, 64), lambda i: (0, 0)),
        ],
        out_specs=[
            pl.BlockSpec((4, BN, 64), lambda i: (0, i, 0)),
            pl.BlockSpec((BN, 4), lambda i: (i, 0)),
            pl.BlockSpec((BN, 4), lambda i: (i, 0)),
        ],
        out_shape=[
            jax.ShapeDtypeStruct((4, NPAD, 64), jnp.float32),
            jax.ShapeDtypeStruct((NPAD, 4), jnp.float32),
            jax.ShapeDtypeStruct((NPAD, 4), jnp.float32),
        ],
    )(xp, w1f, a1l, a1r)


def _dense2_body(ua_ref, ub_ref, sp_ref, w2_ref, a2_ref, z_ref, s2_ref):
    sp = sp_ref[...]
    ssum = sp[:, 0:4] + sp[:, 8:12]
    rinv = jnp.where(ssum > 0, 1.0 / ssum, 0.0)
    ua = ua_ref[...]
    ub = ub_ref[...]
    heads = [ua[0], ub[0], ua[1], ub[1]]
    parts = []
    for hh in range(4):
        v = heads[hh] * rinv[:, hh][:, None]
        parts.append(jnp.where(v > 0, v, jnp.exp(v) - 1.0))
    h = jnp.concatenate(parts, axis=1)
    z2 = jax.lax.dot_general(h, w2_ref[...], (((1,), (1,)), ((), ())),
                             preferred_element_type=jnp.float32)
    z_ref[...] = jnp.stack(
        [z2[:, q * 16:(q + 1) * 16] for q in range(4)], axis=0)
    s2_ref[...] = jax.lax.dot_general(z2, a2_ref[...], (((1,), (1,)), ((), ())),
                                      preferred_element_type=jnp.float32)


def _dense2(out1a, out1b, ssum1p, w2, a2v):
    return pl.pallas_call(
        _dense2_body,
        grid=(NPAD // BN,),
        in_specs=[
            pl.BlockSpec((2, BN, 64), lambda i: (0, i, 0)),
            pl.BlockSpec((2, BN, 64), lambda i: (0, i, 0)),
            pl.BlockSpec((BN, 16), lambda i: (i, 0)),
            pl.BlockSpec((64, 256), lambda i: (0, 0)),
            pl.BlockSpec((2, 64), lambda i: (0, 0)),
        ],
        out_specs=[
            pl.BlockSpec((4, BN, 16), lambda i: (0, i, 0)),
            pl.BlockSpec((BN, 2), lambda i: (i, 0)),
        ],
        out_shape=[
            jax.ShapeDtypeStruct((4, NPAD, 16), jnp.float32),
            jax.ShapeDtypeStruct((NPAD, 2), jnp.float32),
        ],
    )(out1a, out1b, ssum1p, w2, a2v)


def _epilogue_body(oa_ref, ob_ref, sp_ref, out_ref):
    sp = sp_ref[...]
    ssum = sp[:, 0:1] + sp[:, 8:9]
    rinv = jnp.where(ssum > 0, 1.0 / ssum, 0.0)
    oa = oa_ref[...]
    ob = ob_ref[...]
    out_ref[...] = jnp.concatenate(
        [oa[0] * rinv, ob[0] * rinv, oa[1] * rinv, ob[1] * rinv], axis=1)


def _epilogue(out2pa, out2pb, ssum2p):
    return pl.pallas_call(
        _epilogue_body,
        grid=(NPAD // BN,),
        in_specs=[
            pl.BlockSpec((2, BN, 16), lambda i: (0, i, 0)),
            pl.BlockSpec((2, BN, 16), lambda i: (0, i, 0)),
            pl.BlockSpec((BN, 16), lambda i: (i, 0)),
        ],
        out_specs=pl.BlockSpec((BN, 64), lambda i: (i, 0)),
        out_shape=jax.ShapeDtypeStruct((NPAD, 64), jnp.float32),
    )(out2pa, out2pb, ssum2p)


def _make_phase1(E, WH, TS, OFFD):
    B1 = 2000
    ept = E // NC // NS
    nblk = ept // B1
    niter = B1 * WH // 16
    chunk = NPAD // NS
    mesh = plsc.VectorSubcoreMesh(core_axis_name="c", subcore_axis_name="s")

    @functools.partial(
        pl.kernel,
        out_type=[
            jax.ShapeDtypeStruct((E, 8), jnp.float32),
            jax.ShapeDtypeStruct((NPAD, 16), jnp.float32),
        ],
        mesh=mesh,
        compiler_params=pltpu.CompilerParams(use_tc_tiling_on_sc=False, needs_layout_passes=False),
        scratch_types=[
            pltpu.VMEM((NPAD * TS,), jnp.float32),
            pltpu.VMEM((NPAD * TS,), jnp.float32),
            pltpu.VMEM((B1,), jnp.int32),
            pltpu.VMEM((B1,), jnp.int32),
            pltpu.VMEM((B1, 8), jnp.float32),
            pltpu.VMEM_SHARED((NPAD, 8), jnp.float32),
        ],
    )
    def k(ss_hbm, sd_hbm, src_hbm, dst_hbm, w_hbm, ssp_hbm,
          ssv, sdv, srcb, dstb, wb, acc):
        c = lax.axis_index("c")
        s = lax.axis_index("s")
        pltpu.sync_copy(ss_hbm, ssv)
        pltpu.sync_copy(sd_hbm, sdv)
        r0 = s * chunk
        lanes = lax.iota(jnp.int32, 16)
        zvec = lax.full((16,), 0.0, jnp.float32)

        wsh = WH.bit_length() - 1

        def zero_wb(i, carry):
            pvec = i * 16 + lanes
            plsc.store_scatter(
                wb, [lax.shift_right_logical(pvec, 3),
                     lax.bitwise_and(pvec, 7)], zvec)
            return carry

        lax.fori_loop(0, B1 * 8 // 16, zero_wb, 0)
        pltpu.sync_copy(wb.at[pl.ds(0, chunk), :],
                        acc.at[pl.ds(r0, chunk), :])
        plsc.subcore_barrier()
        base0 = c * (E // NC) + s * ept

        def block(b, carry):
            base = base0 + b * B1
            pltpu.sync_copy(src_hbm.at[pl.ds(base, B1)], srcb)
            pltpu.sync_copy(dst_hbm.at[pl.ds(base, B1)], dstb)

            def it(i, carry2):
                pvec = i * 16 + lanes
                jj = lax.shift_right_logical(pvec, wsh)
                rr = lax.bitwise_and(pvec, WH - 1)
                sv = plsc.load_gather(srcb, [jj])
                dv = plsc.load_gather(dstb, [jj])
                ssx = plsc.load_gather(ssv, [sv * TS + rr])
                sdx = plsc.load_gather(sdv, [dv * TS + rr + OFFD])
                e = ssx + sdx
                e = jnp.where(e > 0, e, 0.01 * e)
                plsc.store_scatter(wb, [jj, rr], jnp.exp(e))
                return carry2

            lax.fori_loop(0, niter, it, 0, unroll=2)
            pltpu.sync_copy(wb, acc.at[dstb], add=True)
            pltpu.sync_copy(wb, w_hbm.at[pl.ds(base, B1), :])
            return carry

        lax.fori_loop(0, nblk, block, 0)
        plsc.subcore_barrier()
        pltpu.sync_copy(acc.at[pl.ds(r0, chunk), :],
                        ssp_hbm.at[pl.ds(r0, chunk), pl.ds(c * 8, 8)])

    return k


def _make_phase2(E, HW, B2, callk, l1):
    ept = E // NS
    nblk = ept // B2
    npair = nblk // 2
    assert nblk % 2 == 0 and B2 % 16 == 0
    chunk = NPAD // NS
    mesh = plsc.VectorSubcoreMesh(core_axis_name="c", subcore_axis_name="s")

    @functools.partial(
        pl.kernel,
        out_type=jax.ShapeDtypeStruct((NC, NPAD, HW), jnp.float32),
        mesh=mesh,
        compiler_params=pltpu.CompilerParams(use_tc_tiling_on_sc=False, needs_layout_passes=False),
        scratch_types=[
            pltpu.VMEM((2, 2, B2), jnp.int32),
            pltpu.VMEM((2, B2), jnp.int32),
            pltpu.VMEM((2, B2), jnp.int32),
            pltpu.VMEM((2, B2 * 8), jnp.float32),
            pltpu.VMEM((2, B2, HW), jnp.float32),
            pltpu.VMEM_SHARED((NPAD, HW), jnp.float32),
            pltpu.SemaphoreType.DMA((2,)),
            pltpu.SemaphoreType.DMA((2,)),
        ],
    )
    def k(z_hbm, w_hbm, ei_hbm, out_hbm,
          edb, dstb, idxb, wbuf, rows, acc, semlin, semg):
        c = lax.axis_index("c")
        s = lax.axis_index("s")
        r0 = s * chunk
        zvec = lax.full((16,), 0.0, jnp.float32)

        def zero_rows(j, carry):
            for q in range(HW // 16):
                rows[0, j, pl.ds(q * 16, 16)] = zvec
            return carry

        lax.fori_loop(0, B2, zero_rows, 0)
        nfull = chunk // B2
        for p in range(nfull):
            pltpu.sync_copy(rows.at[0], acc.at[pl.ds(r0 + p * B2, B2), :])
        rem = chunk - nfull * B2
        if rem:
            pltpu.sync_copy(rows.at[0, pl.ds(0, rem), :],
                            acc.at[pl.ds(r0 + nfull * B2, rem), :])
        plsc.subcore_barrier()
        base0 = s * ept
        coff = (2 * c + callk) * NPAD
        hoff = (2 * c + callk) if l1 else 0

        def issue_lin(b, w):
            base = base0 + b * B2
            pltpu.async_copy(ei_hbm.at[:, pl.ds(base, B2)], edb.at[w],
                             semlin.at[w])
            pltpu.async_copy(w_hbm.at[pl.ds(base * 8, B2 * 8)], wbuf.at[w],
                             semlin.at[w])

        def wait_lin(w):
            pltpu.make_async_copy(ei_hbm.at[:, pl.ds(0, B2)], edb.at[w],
                                  semlin.at[w]).wait()
            pltpu.make_async_copy(w_hbm.at[pl.ds(0, B2 * 8)], wbuf.at[w],
                                  semlin.at[w]).wait()

        def issue_gather(w):
            def mkidx(kk, carry):
                sl = pl.ds(kk * 16, 16)
                idxb[w, sl] = edb[w, 0, sl] + coff
                dstb[w, sl] = edb[w, 1, sl]
                return carry

            lax.fori_loop(0, B2 // 16, mkidx, 0, unroll=4)
            pltpu.async_copy(z_hbm.at[idxb.at[w]], rows.at[w], semg.at[w])

        def compute(w):
            pltpu.make_async_copy(z_hbm.at[idxb.at[w]], rows.at[w],
                                  semg.at[w]).wait()

            def scale(j, carry):
                w0 = plsc.load_gather(
                    wbuf.at[w], [jnp.broadcast_to(j * 8 + hoff, (16,))])
                for q in range(HW // 16):
                    rows[w, j, pl.ds(q * 16, 16)] = (
                        rows[w, j, pl.ds(q * 16, 16)] * w0)
                return carry

            lax.fori_loop(0, B2, scale, 0, unroll=4)
            pltpu.sync_copy(rows.at[w], acc.at[dstb.at[w]], add=True)

        issue_lin(0, 0)
        issue_lin(1, 1)
        wait_lin(0)
        issue_gather(0)

        def pair(g, carry):
            b = 2 * g
            for w in range(2):
                compute(w)
                wait_lin(1 - w)
                issue_gather(1 - w)
                issue_lin(b + w + 2, w)
            return carry

        lax.fori_loop(0, npair - 1, pair, 0)
        compute(0)
        wait_lin(1)
        issue_gather(1)
        compute(1)
        plsc.subcore_barrier()
        pltpu.sync_copy(acc.at[pl.ds(r0, chunk), :],
                        out_hbm.at[c, pl.ds(r0, chunk), :])

    return k


def kernel(x, edge_index, W1, a1, W2, a2):
    n, in_dim = x.shape
    E = edge_index.shape[1]
    src = edge_index[0]
    dst = edge_index[1]

    xp = jnp.zeros((NPAD, in_dim), jnp.float32).at[:n].set(x)
    w1f = W1.reshape(4 * 64, in_dim)
    a1l = a1[:, :64]
    a1r = a1[:, 64:]

    z1, ss1, sd1 = _dense1(xp, w1f, a1l, a1r)

    p1a = _make_phase1(E, WH=4, TS=4, OFFD=0)
    w1e, ssum1p = p1a(ss1.reshape(NPAD * 4), sd1.reshape(NPAD * 4), src, dst)

    zf1 = z1.reshape(4 * NPAD, 64)
    wf1 = w1e.reshape(E * 8)
    out1a = _make_phase2(E, HW=64, B2=400, callk=0, l1=True)(
        zf1, wf1, edge_index)
    out1b = _make_phase2(E, HW=64, B2=400, callk=1, l1=True)(
        zf1, wf1, edge_index)

    a2v = jnp.stack([a2[:64], a2[64:]], axis=0)
    z2, s2 = _dense2(out1a, out1b, ssum1p, W2, a2v)

    p1b = _make_phase1(E, WH=1, TS=2, OFFD=1)
    s2f = s2.reshape(NPAD * 2)
    w2e, ssum2p = p1b(s2f, s2f, src, dst)

    zf2 = z2.reshape(4 * NPAD, 16)
    wf2 = w2e.reshape(E * 8)
    out2pa = _make_phase2(E, HW=16, B2=400, callk=0, l1=False)(
        zf2, wf2, edge_index)
    out2pb = _make_phase2(E, HW=16, B2=400, callk=1, l1=False)(
        zf2, wf2, edge_index)

    out = _epilogue(out2pa, out2pb, ssum2p)
    return out[:n]


# speedup vs baseline: 1.3274x; 1.3274x over previous
import functools

import jax
import jax.numpy as jnp
from jax import lax
from jax.experimental import pallas as pl
from jax.experimental.pallas import tpu as pltpu
from jax.experimental.pallas import tpu_sc as plsc

N = 10000
NPAD = 10240
NC = 2
NS = 16
BN = 512

_SC_PARAMS = pltpu.CompilerParams(
    use_tc_tiling_on_sc=False, needs_layout_passes=False)


def _dense1_body(x_ref, w_ref, al_ref, ar_ref, z_ref, ss_ref, sd_ref):
    z = jax.lax.dot_general(x_ref[...], w_ref[...], (((1,), (1,)), ((), ())),
                            preferred_element_type=jnp.float32)
    z_ref[...] = jnp.stack([z[:, :128], z[:, 128:]], axis=0)
    zh = z.reshape(z.shape[0], 4, 64)
    ss_ref[...] = jnp.sum(zh * al_ref[...][None], axis=2)
    sd_ref[...] = jnp.sum(zh * ar_ref[...][None], axis=2)


def _dense1(xp, w1f, a1l, a1r):
    return pl.pallas_call(
        _dense1_body,
        grid=(NPAD // BN,),
        in_specs=[
            pl.BlockSpec((BN, 128), lambda i: (i, 0)),
            pl.BlockSpec((256, 128), lambda i: (0, 0)),
            pl.BlockSpec((4, 64), lambda i: (0, 0)),
            pl.BlockSpec((4, 64), lambda i: (0, 0)),
        ],
        out_specs=[
            pl.BlockSpec((2, BN, 128), lambda i: (0, i, 0)),
            pl.BlockSpec((BN, 4), lambda i: (i, 0)),
            pl.BlockSpec((BN, 4), lambda i: (i, 0)),
        ],
        out_shape=[
            jax.ShapeDtypeStruct((2, NPAD, 128), jnp.float32),
            jax.ShapeDtypeStruct((NPAD, 4), jnp.float32),
            jax.ShapeDtypeStruct((NPAD, 4), jnp.float32),
        ],
    )(xp, w1f, a1l, a1r)


def _dense2_body(u_ref, sp_ref, w2_ref, a2_ref, z_ref, s2_ref):
    sp = sp_ref[...]
    ssum = sp[:, 0:4] + sp[:, 8:12]
    rinv = jnp.where(ssum > 0, 1.0 / ssum, 0.0)
    u = u_ref[...]
    parts = []
    for c in range(2):
        for k in range(2):
            v = u[c, :, k * 64:(k + 1) * 64] * rinv[:, 2 * c + k][:, None]
            parts.append(jnp.where(v > 0, v, jnp.exp(v) - 1.0))
    h = jnp.concatenate(parts, axis=1)
    z2 = jax.lax.dot_general(h, w2_ref[...], (((1,), (1,)), ((), ())),
                             preferred_element_type=jnp.float32)
    z_ref[...] = jnp.stack([z2[:, :32], z2[:, 32:]], axis=0)
    s2_ref[...] = jax.lax.dot_general(z2, a2_ref[...], (((1,), (1,)), ((), ())),
                                      preferred_element_type=jnp.float32)


def _dense2(out1, ssum1p, w2, a2v):
    return pl.pallas_call(
        _dense2_body,
        grid=(NPAD // BN,),
        in_specs=[
            pl.BlockSpec((2, BN, 128), lambda i: (0, i, 0)),
            pl.BlockSpec((BN, 16), lambda i: (i, 0)),
            pl.BlockSpec((64, 256), lambda i: (0, 0)),
            pl.BlockSpec((2, 64), lambda i: (0, 0)),
        ],
        out_specs=[
            pl.BlockSpec((2, BN, 32), lambda i: (0, i, 0)),
            pl.BlockSpec((BN, 2), lambda i: (i, 0)),
        ],
        out_shape=[
            jax.ShapeDtypeStruct((2, NPAD, 32), jnp.float32),
            jax.ShapeDtypeStruct((NPAD, 2), jnp.float32),
        ],
    )(out1, ssum1p, w2, a2v)


def _epilogue_body(o_ref, sp_ref, out_ref):
    sp = sp_ref[...]
    ssum = sp[:, 0:1] + sp[:, 8:9]
    rinv = jnp.where(ssum > 0, 1.0 / ssum, 0.0)
    o = o_ref[...]
    out_ref[...] = jnp.concatenate([o[0] * rinv, o[1] * rinv], axis=1)


def _epilogue(out2p, ssum2p):
    return pl.pallas_call(
        _epilogue_body,
        grid=(NPAD // BN,),
        in_specs=[
            pl.BlockSpec((2, BN, 32), lambda i: (0, i, 0)),
            pl.BlockSpec((BN, 16), lambda i: (i, 0)),
        ],
        out_specs=pl.BlockSpec((BN, 64), lambda i: (i, 0)),
        out_shape=jax.ShapeDtypeStruct((NPAD, 64), jnp.float32),
    )(out2p, ssum2p)


def _make_phase1(E, WH, TS, OFFD):
    B1 = 2000
    ept = E // NC // NS
    nblk = ept // B1
    niter = B1 * WH // 16
    chunk = NPAD // NS
    mesh = plsc.VectorSubcoreMesh(core_axis_name="c", subcore_axis_name="s")

    @functools.partial(
        pl.kernel,
        out_type=[
            jax.ShapeDtypeStruct((E, 8), jnp.float32),
            jax.ShapeDtypeStruct((NPAD, 16), jnp.float32),
        ],
        mesh=mesh,
        compiler_params=_SC_PARAMS,
        scratch_types=[
            pltpu.VMEM((NPAD * TS,), jnp.float32),
            pltpu.VMEM((NPAD * TS,), jnp.float32),
            pltpu.VMEM((B1,), jnp.int32),
            pltpu.VMEM((B1,), jnp.int32),
            pltpu.VMEM((B1, 8), jnp.float32),
            pltpu.VMEM_SHARED((NPAD, 8), jnp.float32),
        ],
    )
    def k(ss_hbm, sd_hbm, src_hbm, dst_hbm, w_hbm, ssp_hbm,
          ssv, sdv, srcb, dstb, wb, acc):
        c = lax.axis_index("c")
        s = lax.axis_index("s")
        pltpu.sync_copy(ss_hbm, ssv)
        pltpu.sync_copy(sd_hbm, sdv)
        r0 = s * chunk
        lanes = lax.iota(jnp.int32, 16)
        zvec = lax.full((16,), 0.0, jnp.float32)
        wsh = WH.bit_length() - 1

        def zero_wb(i, carry):
            pvec = i * 16 + lanes
            plsc.store_scatter(
                wb, [lax.shift_right_logical(pvec, 3),
                     lax.bitwise_and(pvec, 7)], zvec)
            return carry

        lax.fori_loop(0, B1 * 8 // 16, zero_wb, 0)
        pltpu.sync_copy(wb.at[pl.ds(0, chunk), :],
                        acc.at[pl.ds(r0, chunk), :])
        plsc.subcore_barrier()
        base0 = c * (E // NC) + s * ept

        def block(b, carry):
            base = base0 + b * B1
            pltpu.sync_copy(src_hbm.at[pl.ds(base, B1)], srcb)
            pltpu.sync_copy(dst_hbm.at[pl.ds(base, B1)], dstb)

            def it(i, carry2):
                pvec = i * 16 + lanes
                jj = lax.shift_right_logical(pvec, wsh)
                rr = lax.bitwise_and(pvec, WH - 1)
                sv = plsc.load_gather(srcb, [jj])
                dv = plsc.load_gather(dstb, [jj])
                ssx = plsc.load_gather(ssv, [sv * TS + rr])
                sdx = plsc.load_gather(sdv, [dv * TS + rr + OFFD])
                e = ssx + sdx
                e = jnp.where(e > 0, e, 0.01 * e)
                plsc.store_scatter(wb, [jj, rr], jnp.exp(e))
                return carry2

            lax.fori_loop(0, niter, it, 0, unroll=2)
            pltpu.sync_copy(wb, acc.at[dstb], add=True)
            pltpu.sync_copy(wb, w_hbm.at[pl.ds(base, B1), :])
            return carry

        lax.fori_loop(0, nblk, block, 0)
        plsc.subcore_barrier()
        pltpu.sync_copy(acc.at[pl.ds(r0, chunk), :],
                        ssp_hbm.at[pl.ds(r0, chunk), pl.ds(c * 8, 8)])

    return k


def _make_phase2(E, WH, HW, B2):
    ept = E // NS
    nblk = ept // B2
    npair = nblk // 2
    assert nblk % 2 == 0 and B2 % 16 == 0
    chunk = NPAD // NS
    mesh = plsc.VectorSubcoreMesh(core_axis_name="c", subcore_axis_name="s")

    @functools.partial(
        pl.kernel,
        out_type=jax.ShapeDtypeStruct((NC, NPAD, HW), jnp.float32),
        mesh=mesh,
        compiler_params=_SC_PARAMS,
        scratch_types=[
            pltpu.VMEM((2, 2, B2), jnp.int32),
            pltpu.VMEM((2, B2), jnp.int32),
            pltpu.VMEM((2, B2), jnp.int32),
            pltpu.VMEM((2, B2 * 8), jnp.float32),
            pltpu.VMEM((2, B2, HW), jnp.float32),
            pltpu.VMEM_SHARED((NPAD, HW), jnp.float32),
            pltpu.SemaphoreType.DMA((2,)),
            pltpu.SemaphoreType.DMA((2,)),
            pltpu.SemaphoreType.DMA((2,)),
        ],
    )
    def k(z_hbm, w_hbm, ei_hbm, out_hbm,
          edb, dstb, idxb, wbuf, rows, acc, semlin, semg, semsc):
        c = lax.axis_index("c")
        s = lax.axis_index("s")
        r0 = s * chunk
        zvec = lax.full((16,), 0.0, jnp.float32)

        def zero_rows(j, carry):
            for q in range(HW // 16):
                rows[0, j, pl.ds(q * 16, 16)] = zvec
            return carry

        lax.fori_loop(0, B2, zero_rows, 0)
        nfull = chunk // B2
        for p in range(nfull):
            pltpu.sync_copy(rows.at[0], acc.at[pl.ds(r0 + p * B2, B2), :])
        rem = chunk - nfull * B2
        if rem:
            pltpu.sync_copy(rows.at[0, pl.ds(0, rem), :],
                            acc.at[pl.ds(r0 + nfull * B2, rem), :])
        plsc.subcore_barrier()
        base0 = s * ept
        coff = c * NPAD

        def issue_lin(b, w):
            base = base0 + b * B2
            pltpu.async_copy(ei_hbm.at[:, pl.ds(base, B2)], edb.at[w],
                             semlin.at[w])
            pltpu.async_copy(w_hbm.at[pl.ds(base * 8, B2 * 8)], wbuf.at[w],
                             semlin.at[w])

        def wait_lin(w):
            pltpu.make_async_copy(ei_hbm.at[:, pl.ds(0, B2)], edb.at[w],
                                  semlin.at[w]).wait()
            pltpu.make_async_copy(w_hbm.at[pl.ds(0, B2 * 8)], wbuf.at[w],
                                  semlin.at[w]).wait()

        def drain_scatter(w):
            pltpu.make_async_copy(rows.at[w], acc.at[dstb.at[w]],
                                  semsc.at[w]).wait()

        def issue_gather(w, drain):
            if drain:
                drain_scatter(w)

            def mkidx(kk, carry):
                sl = pl.ds(kk * 16, 16)
                idxb[w, sl] = edb[w, 0, sl] + coff
                dstb[w, sl] = edb[w, 1, sl]
                return carry

            lax.fori_loop(0, B2 // 16, mkidx, 0, unroll=4)
            pltpu.async_copy(z_hbm.at[idxb.at[w]], rows.at[w], semg.at[w])

        def compute(w):
            pltpu.make_async_copy(z_hbm.at[idxb.at[w]], rows.at[w],
                                  semg.at[w]).wait()

            def scale(j, carry):
                if WH == 4:
                    i0 = j * 8 + 2 * c
                    w0 = plsc.load_gather(
                        wbuf.at[w], [jnp.broadcast_to(i0, (16,))])
                    w1 = plsc.load_gather(
                        wbuf.at[w], [jnp.broadcast_to(i0 + 1, (16,))])
                    for q in range(8):
                        wq = w0 if q < 4 else w1
                        rows[w, j, pl.ds(q * 16, 16)] = (
                            rows[w, j, pl.ds(q * 16, 16)] * wq)
                else:
                    w0 = plsc.load_gather(
                        wbuf.at[w], [jnp.broadcast_to(j * 8, (16,))])
                    for q in range(HW // 16):
                        rows[w, j, pl.ds(q * 16, 16)] = (
                            rows[w, j, pl.ds(q * 16, 16)] * w0)
                return carry

            lax.fori_loop(0, B2, scale, 0, unroll=4)
            pltpu.async_copy(rows.at[w], acc.at[dstb.at[w]], semsc.at[w],
                             add=True)

        issue_lin(0, 0)
        issue_lin(1, 1)
        wait_lin(0)
        issue_gather(0, drain=False)
        compute(0)
        wait_lin(1)
        issue_gather(1, drain=False)
        issue_lin(2, 0)
        compute(1)
        wait_lin(0)
        issue_gather(0, drain=True)
        issue_lin(3, 1)

        def pair(g, carry):
            b = 2 * g
            for w in range(2):
                compute(w)
                wait_lin(1 - w)
                issue_gather(1 - w, drain=True)
                issue_lin(b + w + 2, w)
            return carry

        lax.fori_loop(1, npair - 1, pair, 0)
        compute(0)
        wait_lin(1)
        issue_gather(1, drain=True)
        compute(1)
        drain_scatter(0)
        drain_scatter(1)
        plsc.subcore_barrier()
        pltpu.sync_copy(acc.at[pl.ds(r0, chunk), :],
                        out_hbm.at[c, pl.ds(r0, chunk), :])

    return k


def kernel(x, edge_index, W1, a1, W2, a2):
    n, in_dim = x.shape
    E = edge_index.shape[1]
    src = edge_index[0]
    dst = edge_index[1]

    xp = jnp.zeros((NPAD, in_dim), jnp.float32).at[:n].set(x)
    w1f = W1.reshape(4 * 64, in_dim)
    a1l = a1[:, :64]
    a1r = a1[:, 64:]

    z1, ss1, sd1 = _dense1(xp, w1f, a1l, a1r)

    p1a = _make_phase1(E, WH=4, TS=4, OFFD=0)
    w1e, ssum1p = p1a(ss1.reshape(NPAD * 4), sd1.reshape(NPAD * 4), src, dst)

    p2a = _make_phase2(E, WH=4, HW=128, B2=80)
    out1 = p2a(z1.reshape(2 * NPAD, 128), w1e.reshape(E * 8), edge_index)

    a2v = jnp.stack([a2[:64], a2[64:]], axis=0)
    z2, s2 = _dense2(out1, ssum1p, W2, a2v)

    p1b = _make_phase1(E, WH=1, TS=2, OFFD=1)
    s2f = s2.reshape(NPAD * 2)
    w2e, ssum2p = p1b(s2f, s2f, src, dst)

    p2b = _make_phase2(E, WH=1, HW=32, B2=400)
    out2p = p2b(z2.reshape(2 * NPAD, 32), w2e.reshape(E * 8), edge_index)

    out = _epilogue(out2p, ssum2p)
    return out[:n]
